# baseline (speedup 1.0000x reference)
import functools

import jax
import jax.numpy as jnp
from jax import lax
from jax.experimental import pallas as pl
from jax.experimental.pallas import tpu as pltpu
from jax.experimental.pallas import tpu_sc as plsc

_S, _N = 60, 4096
_SN = _S * _N
_NARGS = 16
_D = 256
_DW = _D // 2
_AE = 64
_NCMD = 6
_NPAIR = 7
_NG = 9
_A_ROWS = _NCMD * 256 + 256
_B_ROWS = _NPAIR * 65536

_NC, _NS = 2, 16
_NW = _NC * _NS
_P = 128
_NST = 3
_PER_W = _SN // _NW
_NBLK = _PER_W // _P
_NB_TOT = _SN // _P


def _stage1_body(at1_ref, w_ref, cmd_ref, b_ref, ta_ref, tmid_ref):
  at1 = at1_ref[...]
  t0 = jnp.dot(at1, w_ref[pl.ds(0, _AE), :],
               preferred_element_type=jnp.float32)
  t0 = t0 + b_ref[...]
  for c in range(_NCMD):
    ta_ref[pl.ds(c * 256, 256), :] = (
        t0 + cmd_ref[pl.ds(c, 1), :]).astype(jnp.bfloat16)
  t15 = jnp.dot(at1, w_ref[pl.ds(15 * _AE, _AE), :],
                preferred_element_type=jnp.float32)
  ta_ref[pl.ds(_NCMD * 256, 256), :] = t15.astype(jnp.bfloat16)
  for k in range(1, 15):
    tk = jnp.dot(at1, w_ref[pl.ds(k * _AE, _AE), :],
                 preferred_element_type=jnp.float32)
    tmid_ref[k - 1] = tk


def _stage1(arg_table, W, cmd_table, b):
  at1 = arg_table[1:257]
  cmdp = jnp.pad(cmd_table, ((0, 2), (0, 0)))
  return pl.pallas_call(
      _stage1_body,
      out_shape=(jax.ShapeDtypeStruct((_A_ROWS, _D), jnp.bfloat16),
                 jax.ShapeDtypeStruct((14, 256, _D), jnp.float32)),
  )(at1, W, cmdp, b.reshape(1, _D))


def _stage2_body(rowt_ref, colt_ref, out_ref):
  col = colt_ref[0]
  for a_loc in range(32):
    out_ref[pl.ds(a_loc * 256, 256), :] = (
        col + rowt_ref[0, pl.ds(a_loc, 1), :]).astype(jnp.bfloat16)


def _stage2(tmid):
  return pl.pallas_call(
      _stage2_body,
      grid=(_NPAIR, 8),
      in_specs=[
          pl.BlockSpec((1, 32, _D), lambda j, i: (2 * j, i, 0)),
          pl.BlockSpec((1, 256, _D), lambda j, i: (2 * j + 1, 0, 0)),
      ],
      out_specs=pl.BlockSpec((32 * 256, _D), lambda j, i: (j * 8 + i, 0)),
      out_shape=jax.ShapeDtypeStruct((_B_ROWS, _D), jnp.bfloat16),
  )(tmid, tmid)


def _pack_words(t_bf16):
  rows = t_bf16.shape[0]
  return lax.bitcast_convert_type(
      t_bf16.reshape(rows, _DW, 2), jnp.int32)


def _unpack_fold(acc_v, st_v, first):
  def row(r, carry):
    for t in range(_DW // 16):
      w = st_v[r, pl.ds(t * 16, 16)]
      lo = lax.bitcast_convert_type(
          lax.shift_left(w, jnp.int32(16)), jnp.float32)
      hi = lax.bitcast_convert_type(
          lax.bitwise_and(w, jnp.int32(-65536)), jnp.float32)
      if first:
        acc_v[r, pl.ds(t * 16, 16)] = lo
        acc_v[r, pl.ds(_DW + t * 16, 16)] = hi
      else:
        plsc.addupdate(acc_v.at[r, pl.ds(t * 16, 16)], lo)
        plsc.addupdate(acc_v.at[r, pl.ds(_DW + t * 16, 16)], hi)
    return carry
  lax.fori_loop(0, _P, row, 0)


def _sc_body(slab_hbm, ta_hbm, tb_hbm, out_hbm, raw_v, idx_v, acc_v, *rest):
  sts = rest[:_NST]
  sems = rest[_NST:]
  wid = lax.axis_index("s") * _NC + lax.axis_index("c")
  tabs = [ta_hbm] + [tb_hbm] * _NPAIR + [ta_hbm]

  def block(j, carry):
    bidx = wid * _NBLK + j
    base = bidx * _P
    pltpu.sync_copy(slab_hbm.at[bidx], raw_v)
    for t in range(_P // 16):
      sl = pl.ds(t * 16, 16)
      idx_v[0, sl] = raw_v[0, sl] * 256 + raw_v[1, sl]
      for g in range(1, 1 + _NPAIR):
        idx_v[g, sl] = (raw_v[2 * g, sl] * 256 + raw_v[2 * g + 1, sl]
                        + (g - 1) * 65536)
      idx_v[_NG - 1, sl] = raw_v[16, sl] + _NCMD * 256
    descs = {}
    for g in range(min(_NST - 1, _NG)):
      descs[g] = pltpu.async_copy(
          tabs[g].at[idx_v.at[g]], sts[g % _NST], sems[g % _NST])
    for g in range(_NG):
      nxt = g + _NST - 1
      if nxt < _NG:
        descs[nxt] = pltpu.async_copy(
            tabs[nxt].at[idx_v.at[nxt]], sts[nxt % _NST], sems[nxt % _NST])
      descs[g].wait()
      _unpack_fold(acc_v, sts[g % _NST], first=(g == 0))
    pltpu.sync_copy(acc_v, out_hbm.at[pl.ds(base, _P)])
    return carry

  lax.fori_loop(0, _NBLK, block, 0)


def _sc_gather_sum(slab, table_a, table_b):
  mesh = plsc.VectorSubcoreMesh(core_axis_name="c", subcore_axis_name="s")
  f = pl.kernel(
      _sc_body,
      out_type=jax.ShapeDtypeStruct((_SN, _D), jnp.float32),
      mesh=mesh,
      scratch_types=[
          pltpu.VMEM((_NARGS + 1, _P), jnp.int32),
          pltpu.VMEM((_NG, _P), jnp.int32),
          pltpu.VMEM((_P, _D), jnp.float32),
      ] + [pltpu.VMEM((_P, _DW), jnp.int32) for _ in range(_NST)]
        + [pltpu.SemaphoreType.DMA for _ in range(_NST)],
  )
  return f(slab, table_a, table_b)


def kernel(commands, args, cmd_table, arg_table, W, b):
  sigma = jnp.stack(
      [jnp.arange(_DW, dtype=jnp.int32),
       jnp.arange(_DW, dtype=jnp.int32) + _DW], axis=1).reshape(_D)
  table_a, tmid = _stage1(arg_table, W[:, sigma], cmd_table[:, sigma], b[sigma])
  table_b = _stage2(tmid)
  flat = jnp.concatenate(
      [commands.reshape(_SN, 1), args.reshape(_SN, _NARGS)], axis=1)
  slab = flat.reshape(_NB_TOT, _P, _NARGS + 1).swapaxes(1, 2)
  out = _sc_gather_sum(slab, _pack_words(table_a), _pack_words(table_b))
  return out.reshape(_S, _N, _D)


# speedup vs baseline: 1.4341x; 1.4341x over previous
import functools

import jax
import jax.numpy as jnp
from jax import lax
from jax.experimental import pallas as pl
from jax.experimental.pallas import tpu as pltpu
from jax.experimental.pallas import tpu_sc as plsc

_S, _N = 60, 4096
_SN = _S * _N
_NARGS = 16
_D = 256
_DW = _D // 2
_AE = 64
_NCMD = 6
_NPAIR = 7
_NG = 9
_A_ROWS = _NCMD * 256 + 256
_B_ROWS = _NPAIR * 65536

_NC, _NS = 2, 16
_NW = _NC * _NS
_P = 128
_NST = 3
_PER_W = _SN // _NW
_NBLK = _PER_W // _P
_NB_TOT = _SN // _P


def _stage1_body(at1_ref, w_ref, cmd_ref, b_ref, ta_ref, tmid_ref):
  at1 = at1_ref[...]
  t0 = jnp.dot(at1, w_ref[pl.ds(0, _AE), :],
               preferred_element_type=jnp.float32)
  t0 = t0 + b_ref[...]
  for c in range(_NCMD):
    ta_ref[pl.ds(c * 256, 256), :] = (
        t0 + cmd_ref[pl.ds(c, 1), :]).astype(jnp.bfloat16)
  t15 = jnp.dot(at1, w_ref[pl.ds(15 * _AE, _AE), :],
                preferred_element_type=jnp.float32)
  ta_ref[pl.ds(_NCMD * 256, 256), :] = t15.astype(jnp.bfloat16)
  for k in range(1, 15):
    tk = jnp.dot(at1, w_ref[pl.ds(k * _AE, _AE), :],
                 preferred_element_type=jnp.float32)
    tmid_ref[k - 1] = tk


def _stage1(arg_table, W, cmd_table, b):
  at1 = arg_table[1:257]
  cmdp = jnp.pad(cmd_table, ((0, 2), (0, 0)))
  return pl.pallas_call(
      _stage1_body,
      out_shape=(jax.ShapeDtypeStruct((_A_ROWS, _D), jnp.bfloat16),
                 jax.ShapeDtypeStruct((14, 256, _D), jnp.float32)),
  )(at1, W, cmdp, b.reshape(1, _D))


def _stage2_body(rowt_ref, colt_ref, out_ref):
  col = colt_ref[0]
  for a_loc in range(32):
    out_ref[pl.ds(a_loc * 256, 256), :] = (
        col + rowt_ref[0, pl.ds(a_loc, 1), :]).astype(jnp.bfloat16)


def _stage2(tmid):
  return pl.pallas_call(
      _stage2_body,
      grid=(_NPAIR, 8),
      in_specs=[
          pl.BlockSpec((1, 32, _D), lambda j, i: (2 * j, i, 0)),
          pl.BlockSpec((1, 256, _D), lambda j, i: (2 * j + 1, 0, 0)),
      ],
      out_specs=pl.BlockSpec((32 * 256, _D), lambda j, i: (j * 8 + i, 0)),
      out_shape=jax.ShapeDtypeStruct((_B_ROWS, _D), jnp.bfloat16),
  )(tmid, tmid)


def _pack_words(t_bf16):
  rows = t_bf16.shape[0]
  return lax.bitcast_convert_type(
      t_bf16.reshape(rows, _DW, 2), jnp.int32)


def _unpack_fold(acc_v, st_v, first):
  @plsc.parallel_loop(0, _P, unroll=4)
  def row(r):
    for t in range(_DW // 16):
      w = st_v[r, pl.ds(t * 16, 16)]
      lo = lax.bitcast_convert_type(
          lax.shift_left(w, jnp.int32(16)), jnp.float32)
      hi = lax.bitcast_convert_type(
          lax.bitwise_and(w, jnp.int32(-65536)), jnp.float32)
      if first:
        acc_v[r, pl.ds(t * 16, 16)] = lo
        acc_v[r, pl.ds(_DW + t * 16, 16)] = hi
      else:
        plsc.addupdate(acc_v.at[r, pl.ds(t * 16, 16)], lo)
        plsc.addupdate(acc_v.at[r, pl.ds(_DW + t * 16, 16)], hi)


def _sc_body(slab_hbm, ta_hbm, tb_hbm, out_hbm, raw_v, idx_v, acc_v, *rest):
  sts = rest[:_NST]
  sems = rest[_NST:]
  wid = lax.axis_index("s") * _NC + lax.axis_index("c")
  tabs = [ta_hbm] + [tb_hbm] * _NPAIR + [ta_hbm]

  def block(j, carry):
    bidx = wid * _NBLK + j
    base = bidx * _P
    pltpu.sync_copy(slab_hbm.at[bidx], raw_v)
    for t in range(_P // 16):
      sl = pl.ds(t * 16, 16)
      idx_v[0, sl] = raw_v[0, sl] * 256 + raw_v[1, sl]
      for g in range(1, 1 + _NPAIR):
        idx_v[g, sl] = (raw_v[2 * g, sl] * 256 + raw_v[2 * g + 1, sl]
                        + (g - 1) * 65536)
      idx_v[_NG - 1, sl] = raw_v[16, sl] + _NCMD * 256
    descs = {}
    for g in range(min(_NST - 1, _NG)):
      descs[g] = pltpu.async_copy(
          tabs[g].at[idx_v.at[g]], sts[g % _NST], sems[g % _NST])
    for g in range(_NG):
      nxt = g + _NST - 1
      if nxt < _NG:
        descs[nxt] = pltpu.async_copy(
            tabs[nxt].at[idx_v.at[nxt]], sts[nxt % _NST], sems[nxt % _NST])
      descs[g].wait()
      _unpack_fold(acc_v, sts[g % _NST], first=(g == 0))
    pltpu.sync_copy(acc_v, out_hbm.at[pl.ds(base, _P)])
    return carry

  lax.fori_loop(0, _NBLK, block, 0)


def _sc_gather_sum(slab, table_a, table_b):
  mesh = plsc.VectorSubcoreMesh(core_axis_name="c", subcore_axis_name="s")
  f = pl.kernel(
      _sc_body,
      out_type=jax.ShapeDtypeStruct((_SN, _D), jnp.float32),
      mesh=mesh,
      scratch_types=[
          pltpu.VMEM((_NARGS + 1, _P), jnp.int32),
          pltpu.VMEM((_NG, _P), jnp.int32),
          pltpu.VMEM((_P, _D), jnp.float32),
      ] + [pltpu.VMEM((_P, _DW), jnp.int32) for _ in range(_NST)]
        + [pltpu.SemaphoreType.DMA for _ in range(_NST)],
  )
  return f(slab, table_a, table_b)


def kernel(commands, args, cmd_table, arg_table, W, b):
  sigma = jnp.stack(
      [jnp.arange(_DW, dtype=jnp.int32),
       jnp.arange(_DW, dtype=jnp.int32) + _DW], axis=1).reshape(_D)
  table_a, tmid = _stage1(arg_table, W[:, sigma], cmd_table[:, sigma], b[sigma])
  table_b = _stage2(tmid)
  flat = jnp.concatenate(
      [commands.reshape(_SN, 1), args.reshape(_SN, _NARGS)], axis=1)
  slab = flat.reshape(_NB_TOT, _P, _NARGS + 1).swapaxes(1, 2)
  out = _sc_gather_sum(slab, _pack_words(table_a), _pack_words(table_b))
  return out.reshape(_S, _N, _D)


# speedup vs baseline: 1.7194x; 1.1989x over previous
import functools

import jax
import jax.numpy as jnp
from jax import lax
from jax.experimental import pallas as pl
from jax.experimental.pallas import tpu as pltpu
from jax.experimental.pallas import tpu_sc as plsc

_S, _N = 60, 4096
_SN = _S * _N
_NARGS = 16
_D = 256
_DW = _D // 2
_AE = 64
_NCMD = 6
_NPAIR = 7
_NG = 9
_A_ROWS = _NCMD * 256 + 256
_B_ROWS = _NPAIR * 65536

_NC, _NS = 2, 16
_NW = _NC * _NS
_P = 80
_H = _P // 2
_PER_W = _SN // _NW
_NBLK = _PER_W // _P
_NB_TOT = _SN // _P


def _stage1_body(at1_ref, w_ref, cmd_ref, b_ref, ta_ref, tmid_ref):
  at1 = at1_ref[...]
  t0 = jnp.dot(at1, w_ref[pl.ds(0, _AE), :],
               preferred_element_type=jnp.float32)
  t0 = t0 + b_ref[...]
  for c in range(_NCMD):
    ta_ref[pl.ds(c * 256, 256), :] = (
        t0 + cmd_ref[pl.ds(c, 1), :]).astype(jnp.bfloat16)
  t15 = jnp.dot(at1, w_ref[pl.ds(15 * _AE, _AE), :],
                preferred_element_type=jnp.float32)
  ta_ref[pl.ds(_NCMD * 256, 256), :] = t15.astype(jnp.bfloat16)
  for k in range(1, 15):
    tk = jnp.dot(at1, w_ref[pl.ds(k * _AE, _AE), :],
                 preferred_element_type=jnp.float32)
    tmid_ref[k - 1] = tk


def _stage1(arg_table, W, cmd_table, b):
  at1 = arg_table[1:257]
  cmdp = jnp.pad(cmd_table, ((0, 2), (0, 0)))
  return pl.pallas_call(
      _stage1_body,
      out_shape=(jax.ShapeDtypeStruct((_A_ROWS, _D), jnp.bfloat16),
                 jax.ShapeDtypeStruct((14, 256, _D), jnp.float32)),
  )(at1, W, cmdp, b.reshape(1, _D))


def _stage2_body(rowt_ref, colt_ref, out_ref):
  col = colt_ref[0]
  for a_loc in range(32):
    out_ref[pl.ds(a_loc * 256, 256), :] = (
        col + rowt_ref[0, pl.ds(a_loc, 1), :]).astype(jnp.bfloat16)


def _stage2(tmid):
  return pl.pallas_call(
      _stage2_body,
      grid=(_NPAIR, 8),
      in_specs=[
          pl.BlockSpec((1, 32, _D), lambda j, i: (2 * j, i, 0)),
          pl.BlockSpec((1, 256, _D), lambda j, i: (2 * j + 1, 0, 0)),
      ],
      out_specs=pl.BlockSpec((32 * 256, _D), lambda j, i: (j * 8 + i, 0)),
      out_shape=jax.ShapeDtypeStruct((_B_ROWS, _D), jnp.bfloat16),
  )(tmid, tmid)


def _pack_words(t_bf16):
  rows = t_bf16.shape[0]
  return lax.bitcast_convert_type(t_bf16.reshape(rows, _DW, 2), jnp.int32)


def _build_idx(raw_v, idx_v):
  for t in range(_P // 16):
    sl = pl.ds(t * 16, 16)
    idx_v[0, sl] = raw_v[0, sl] * 256 + raw_v[1, sl]
    for g in range(1, 1 + _NPAIR):
      idx_v[g, sl] = (raw_v[2 * g, sl] * 256 + raw_v[2 * g + 1, sl]
                      + (g - 1) * 65536)
    idx_v[_NG - 1, sl] = raw_v[16, sl] + _NCMD * 256


def _fold(sts, out_v):
  @plsc.parallel_loop(0, _H, unroll=2)
  def row(r):
    for t in range(_DW // 16):
      sl = pl.ds(t * 16, 16)
      w = sts[0][r, sl]
      lo = lax.shift_left(w, jnp.int32(16))
      hi = lax.bitwise_and(w, jnp.int32(-65536))
      acc_lo = lax.bitcast_convert_type(lo, jnp.float32)
      acc_hi = lax.bitcast_convert_type(hi, jnp.float32)
      for g in range(1, _NG):
        w = sts[g][r, sl]
        lo = lax.shift_left(w, jnp.int32(16))
        hi = lax.bitwise_and(w, jnp.int32(-65536))
        acc_lo = acc_lo + lax.bitcast_convert_type(lo, jnp.float32)
        acc_hi = acc_hi + lax.bitcast_convert_type(hi, jnp.float32)
      out_v[r, sl] = acc_lo
      out_v[r, pl.ds(_DW + t * 16, 16)] = acc_hi


def _sc_body(slab_hbm, ta_hbm, tb_hbm, out_hbm, raw_a, raw_b, idx_a, idx_b,
             out0_v, out1_v, *rest):
  st0 = rest[:_NG]
  st1 = rest[_NG:2 * _NG]
  sem_s0, sem_s1, sem_o0, sem_o1 = rest[2 * _NG:]
  wid = lax.axis_index("s") * _NC + lax.axis_index("c")
  tabs = [ta_hbm] + [tb_hbm] * _NPAIR + [ta_hbm]
  wbase = wid * _NBLK

  def fire9(idx_v, h, sts, sem):
    return [pltpu.async_copy(
        tabs[g].at[idx_v.at[g, pl.ds(h * _H, _H)]], sts[g], sem)
        for g in range(_NG)]

  def drain9(sts, sem):
    for g in range(_NG):
      pltpu.make_async_copy(tb_hbm.at[pl.ds(0, _H)], sts[g], sem).wait()

  def drain_out(out_v, sem):
    pltpu.make_async_copy(out_v, out_hbm.at[pl.ds(0, _H)], sem).wait()

  pltpu.async_copy(out_hbm.at[pl.ds(wbase * _P, _H)], out0_v, sem_o0)
  pltpu.async_copy(out_hbm.at[pl.ds(wbase * _P, _H)], out1_v, sem_o1)
  pltpu.sync_copy(slab_hbm.at[wbase], raw_a)
  _build_idx(raw_a, idx_a)
  fire9(idx_a, 0, st0, sem_s0)

  def body(j2, carry):
    b0 = wbase + 2 * j2
    b1 = b0 + 1
    b2 = wbase + jnp.minimum(2 * j2 + 2, _NBLK - 1)
    d_s1 = fire9(idx_a, 1, st1, sem_s1)
    drain9(st0, sem_s0)
    drain_out(out0_v, sem_o0)
    _fold(st0, out0_v)
    d_o0 = pltpu.async_copy(out0_v, out_hbm.at[pl.ds(b0 * _P, _H)], sem_o0)
    pltpu.sync_copy(slab_hbm.at[b1], raw_b)
    _build_idx(raw_b, idx_b)
    d_s0 = fire9(idx_b, 0, st0, sem_s0)
    for d in d_s1:
      d.wait()
    drain_out(out1_v, sem_o1)
    _fold(st1, out1_v)
    d_o1 = pltpu.async_copy(
        out1_v, out_hbm.at[pl.ds(b0 * _P + _H, _H)], sem_o1)
    for d in d_s0:
      d.wait()
    d_s1b = fire9(idx_b, 1, st1, sem_s1)
    d_o0.wait()
    _fold(st0, out0_v)
    pltpu.async_copy(out0_v, out_hbm.at[pl.ds(b1 * _P, _H)], sem_o0)
    pltpu.sync_copy(slab_hbm.at[b2], raw_a)
    _build_idx(raw_a, idx_a)
    fire9(idx_a, 0, st0, sem_s0)
    for d in d_s1b:
      d.wait()
    d_o1.wait()
    _fold(st1, out1_v)
    pltpu.async_copy(out1_v, out_hbm.at[pl.ds(b1 * _P + _H, _H)], sem_o1)
    return carry

  lax.fori_loop(0, _NBLK // 2, body, 0)
  drain9(st0, sem_s0)
  drain_out(out0_v, sem_o0)
  drain_out(out1_v, sem_o1)


def _sc_gather_sum(slab, table_a, table_b):
  mesh = plsc.VectorSubcoreMesh(core_axis_name="c", subcore_axis_name="s")
  f = pl.kernel(
      _sc_body,
      out_type=jax.ShapeDtypeStruct((_SN, _D), jnp.float32),
      mesh=mesh,
      scratch_types=[
          pltpu.VMEM((_NARGS + 1, _P), jnp.int32),
          pltpu.VMEM((_NARGS + 1, _P), jnp.int32),
          pltpu.VMEM((_NG, _P), jnp.int32),
          pltpu.VMEM((_NG, _P), jnp.int32),
          pltpu.VMEM((_H, _D), jnp.float32),
          pltpu.VMEM((_H, _D), jnp.float32),
      ] + [pltpu.VMEM((_H, _DW), jnp.int32) for _ in range(2 * _NG)]
        + [pltpu.SemaphoreType.DMA for _ in range(4)],
  )
  return f(slab, table_a, table_b)


def kernel(commands, args, cmd_table, arg_table, W, b):
  sigma = jnp.stack(
      [jnp.arange(_DW, dtype=jnp.int32),
       jnp.arange(_DW, dtype=jnp.int32) + _DW], axis=1).reshape(_D)
  table_a, tmid = _stage1(arg_table, W[:, sigma], cmd_table[:, sigma], b[sigma])
  table_b = _stage2(tmid)
  flat = jnp.concatenate(
      [commands.reshape(_SN, 1), args.reshape(_SN, _NARGS)], axis=1)
  slab = flat.reshape(_NB_TOT, _P, _NARGS + 1).swapaxes(1, 2)
  out = _sc_gather_sum(slab, _pack_words(table_a), _pack_words(table_b))
  return out.reshape(_S, _N, _D)
